# baseline (speedup 1.0000x reference)
import functools

import jax
import jax.numpy as jnp
from jax import lax
from jax.experimental import pallas as pl
from jax.experimental.pallas import tpu as pltpu
from jax.experimental.pallas import tpu_sc as plsc

N_KF = 16
N_MP = 32
M = 65536
FX = 320.0
FY = 320.0
CX = 320.0
CY = 240.0

NUM_WORKERS = 32
MEAS_PER_W = M // NUM_WORKERS
N_TABLE = N_KF * N_MP
TT_KF = N_KF * 16
TT_LEN = TT_KF + N_MP * 3


def _sc_body(ids_hbm, tt_hbm, idx_hbm, x_hbm, y_hbm,
             ids_v, x_v, y_v, tabx_v, taby_v, tt_v,
             idx_v, invkf_v, invmp_v,
             sem_idx, sem_tab, sem_meas):
    wid = lax.axis_index("s") * 2 + lax.axis_index("c")
    iota = lax.iota(jnp.int32, 16)

    cp_ids = pltpu.async_copy(ids_hbm.at[pl.ds(wid * (2 * MEAS_PER_W), 2 * MEAS_PER_W)], ids_v, sem_meas)
    cp_idx = pltpu.async_copy(idx_hbm, idx_v, sem_idx)
    cp_tt = pltpu.async_copy(tt_hbm, tt_v, sem_tab)

    cp_idx.wait()
    plsc.store_scatter(invkf_v, [idx_v[pl.ds(0, 16)]], iota)
    plsc.store_scatter(invmp_v, [idx_v[pl.ds(16, 16)]], iota)
    plsc.store_scatter(invmp_v, [idx_v[pl.ds(32, 16)]], iota + 16)
    cp_tt.wait()

    invkf_vec = invkf_v[...]
    for t in range(N_TABLE // 16):
        cbase = 16 * t
        kid = cbase >> 5
        kpos = invkf_vec[kid]
        kbase = kpos * 16
        pose = tt_v[pl.ds(kbase, 16)]
        r = [pose[j] for j in range(12)]
        mid = (cbase & 31) + iota
        mpos = plsc.load_gather(invmp_v, [mid])
        mbase = TT_KF + mpos + lax.shift_left(mpos, jnp.int32(1))
        px = plsc.load_gather(tt_v, [mbase])
        py = plsc.load_gather(tt_v, [mbase + 1])
        pz = plsc.load_gather(tt_v, [mbase + 2])
        x = r[0] * px + r[1] * py + r[2] * pz + r[3]
        y = r[4] * px + r[5] * py + r[6] * pz + r[7]
        z = r[8] * px + r[9] * py + r[10] * pz + r[11]
        inv = jnp.float32(1.0) / z
        ptx = x * inv * jnp.float32(FX) + jnp.float32(CX)
        pty = y * inv * jnp.float32(FY) + jnp.float32(CY)
        tabx_v[pl.ds(cbase, 16)] = ptx
        taby_v[pl.ds(cbase, 16)] = pty

    cp_ids.wait()

    def gather_block(_, boff):
        xoff = lax.shift_right_logical(boff, jnp.int32(1))
        for j in range(0, 128, 16):
            kf = ids_v[pl.ds(boff + j, 16)]
            mp = ids_v[pl.ds(boff + 128 + j, 16)]
            c = plsc.bitcast(lax.shift_left(kf, jnp.uint32(5)) + mp, jnp.int32)
            x_v[pl.ds(xoff + j, 16)] = plsc.load_gather(tabx_v, [c])
            y_v[pl.ds(xoff + j, 16)] = plsc.load_gather(taby_v, [c])
        return boff + jnp.int32(256)

    lax.fori_loop(0, MEAS_PER_W // 128, gather_block, jnp.int32(0), unroll=2)

    pltpu.sync_copy(x_v, x_hbm.at[pl.ds(wid * MEAS_PER_W, MEAS_PER_W)])
    pltpu.sync_copy(y_v, y_hbm.at[pl.ds(wid * MEAS_PER_W, MEAS_PER_W)])


def kernel(tMP, tKF, measurements, idxMP, idxKF):
    meas32 = measurements.astype(jnp.uint32)
    ids_flat = meas32.reshape(M // 128, 128, 2).transpose(0, 2, 1).reshape(2 * M)
    pose_blocks = jnp.pad(tKF[:, :3, :].reshape(N_KF, 12), ((0, 0), (0, 4)))
    tt = jnp.concatenate(
        [pose_blocks.reshape(TT_KF), tMP.reshape(N_MP * 3)]
    ).astype(jnp.float32)
    idx = jnp.concatenate([idxKF, idxMP])

    mesh = plsc.VectorSubcoreMesh(core_axis_name="c", subcore_axis_name="s")
    sc_call = functools.partial(
        pl.kernel,
        mesh=mesh,
        out_type=(
            jax.ShapeDtypeStruct((M,), jnp.float32),
            jax.ShapeDtypeStruct((M,), jnp.float32),
        ),
        compiler_params=pltpu.CompilerParams(needs_layout_passes=False),
        scratch_types=[
            pltpu.VMEM((2 * MEAS_PER_W,), jnp.uint32),
            pltpu.VMEM((MEAS_PER_W,), jnp.float32),
            pltpu.VMEM((MEAS_PER_W,), jnp.float32),
            pltpu.VMEM((N_TABLE,), jnp.float32),
            pltpu.VMEM((N_TABLE,), jnp.float32),
            pltpu.VMEM((TT_LEN,), jnp.float32),
            pltpu.VMEM((N_KF + N_MP,), jnp.int32),
            pltpu.VMEM((N_KF,), jnp.int32),
            pltpu.VMEM((N_MP,), jnp.int32),
            pltpu.SemaphoreType.DMA,
            pltpu.SemaphoreType.DMA,
            pltpu.SemaphoreType.DMA,
        ],
    )(_sc_body)
    out_x, out_y = sc_call(ids_flat, tt, idx)
    obs2d = jnp.stack([out_x, out_y], axis=1).astype(jnp.float64)
    return obs2d


# speedup vs baseline: 1.0969x; 1.0969x over previous
import functools

import jax
import jax.numpy as jnp
from jax import lax
from jax.experimental import pallas as pl
from jax.experimental.pallas import tpu as pltpu
from jax.experimental.pallas import tpu_sc as plsc

N_KF = 16
N_MP = 32
M = 65536
FX = 320.0
FY = 320.0
CX = 320.0
CY = 240.0

NUM_WORKERS = 32
MEAS_PER_W = M // NUM_WORKERS
N_TABLE = N_KF * N_MP
TT_KF = N_KF * 16
TT_LEN = TT_KF + N_MP * 3


def _sc_body(ids_hbm, tt_hbm, idx_hbm, x_hbm, y_hbm,
             ids_v, x_v, y_v, tabx_v, taby_v, tt_v,
             idx_v, invkf_v, invmp_v,
             sem_idx, sem_tab, sem_meas):
    wid = lax.axis_index("s") * 2 + lax.axis_index("c")
    iota = lax.iota(jnp.int32, 16)

    cp_ids = pltpu.async_copy(ids_hbm.at[pl.ds(wid * (2 * MEAS_PER_W), 2 * MEAS_PER_W)], ids_v, sem_meas)
    cp_idx = pltpu.async_copy(idx_hbm, idx_v, sem_idx)
    cp_tt = pltpu.async_copy(tt_hbm, tt_v, sem_tab)

    cp_idx.wait()
    plsc.store_scatter(invkf_v, [idx_v[pl.ds(0, 16)]], iota)
    plsc.store_scatter(invmp_v, [idx_v[pl.ds(16, 16)]], iota)
    plsc.store_scatter(invmp_v, [idx_v[pl.ds(32, 16)]], iota + 16)
    cp_tt.wait()

    invkf_vec = invkf_v[...]
    for t in range(N_TABLE // 16):
        cbase = 16 * t
        kid = cbase >> 5
        kpos = invkf_vec[kid]
        kbase = kpos * 16
        pose = tt_v[pl.ds(kbase, 16)]
        r = [pose[j] for j in range(12)]
        mid = (cbase & 31) + iota
        mpos = plsc.load_gather(invmp_v, [mid])
        mbase = TT_KF + mpos + lax.shift_left(mpos, jnp.int32(1))
        px = plsc.load_gather(tt_v, [mbase])
        py = plsc.load_gather(tt_v, [mbase + 1])
        pz = plsc.load_gather(tt_v, [mbase + 2])
        x = r[0] * px + r[1] * py + r[2] * pz + r[3]
        y = r[4] * px + r[5] * py + r[6] * pz + r[7]
        z = r[8] * px + r[9] * py + r[10] * pz + r[11]
        inv = jnp.float32(1.0) / z
        ptx = x * inv * jnp.float32(FX) + jnp.float32(CX)
        pty = y * inv * jnp.float32(FY) + jnp.float32(CY)
        tabx_v[pl.ds(cbase, 16)] = ptx
        taby_v[pl.ds(cbase, 16)] = pty

    cp_ids.wait()

    def gather_block(_, boff):
        xoff = lax.shift_right_logical(boff, jnp.int32(1))
        for j in range(0, 128, 16):
            kf = ids_v[pl.ds(boff + j, 16)]
            mp = ids_v[pl.ds(boff + 128 + j, 16)]
            c = lax.shift_left(kf, jnp.int32(5)) + mp
            x_v[pl.ds(xoff + j, 16)] = plsc.load_gather(tabx_v, [c])
            y_v[pl.ds(xoff + j, 16)] = plsc.load_gather(taby_v, [c])
        return boff + jnp.int32(256)

    lax.fori_loop(0, MEAS_PER_W // 128, gather_block, jnp.int32(0), unroll=2)

    pltpu.sync_copy(x_v, x_hbm.at[pl.ds(wid * MEAS_PER_W, MEAS_PER_W)])
    pltpu.sync_copy(y_v, y_hbm.at[pl.ds(wid * MEAS_PER_W, MEAS_PER_W)])


def kernel(tMP, tKF, measurements, idxMP, idxKF):
    meas32 = measurements.astype(jnp.int32)
    ids_flat = meas32.reshape(M // 128, 128, 2).transpose(0, 2, 1).reshape(2 * M)
    pose_blocks = jnp.pad(tKF[:, :3, :].reshape(N_KF, 12), ((0, 0), (0, 4)))
    tt = jnp.concatenate(
        [pose_blocks.reshape(TT_KF), tMP.reshape(N_MP * 3)]
    ).astype(jnp.float32)
    idx = jnp.concatenate([idxKF, idxMP])

    mesh = plsc.VectorSubcoreMesh(core_axis_name="c", subcore_axis_name="s")
    sc_call = functools.partial(
        pl.kernel,
        mesh=mesh,
        out_type=(
            jax.ShapeDtypeStruct((M,), jnp.float32),
            jax.ShapeDtypeStruct((M,), jnp.float32),
        ),
        compiler_params=pltpu.CompilerParams(needs_layout_passes=False),
        scratch_types=[
            pltpu.VMEM((2 * MEAS_PER_W,), jnp.int32),
            pltpu.VMEM((MEAS_PER_W,), jnp.float32),
            pltpu.VMEM((MEAS_PER_W,), jnp.float32),
            pltpu.VMEM((N_TABLE,), jnp.float32),
            pltpu.VMEM((N_TABLE,), jnp.float32),
            pltpu.VMEM((TT_LEN,), jnp.float32),
            pltpu.VMEM((N_KF + N_MP,), jnp.int32),
            pltpu.VMEM((N_KF,), jnp.int32),
            pltpu.VMEM((N_MP,), jnp.int32),
            pltpu.SemaphoreType.DMA,
            pltpu.SemaphoreType.DMA,
            pltpu.SemaphoreType.DMA,
        ],
    )(_sc_body)
    out_x, out_y = sc_call(ids_flat, tt, idx)
    obs2d = jnp.stack([out_x, out_y], axis=1).astype(jnp.float64)
    return obs2d
